# Initial kernel scaffold; baseline (speedup 1.0000x reference)
import jax
import jax.numpy as jnp
from jax.experimental import pallas as pl


def kernel(x, edge_index, batch, additional_input, W1, b1, W2, b2, w_pool, b_pool, W_go, b_go, W_f, b_f):
    raise NotImplementedError("write your pallas kernel here")


# speedup vs baseline: 8.0921x; 8.0921x over previous
import functools

import jax
import jax.numpy as jnp
from jax import lax
from jax.experimental import pallas as pl
from jax.experimental.pallas import tpu as pltpu
from jax.experimental.pallas import tpu_sc as plsc

N = 10000
E = 320000
D = 128
G = 64
CHUNK = 128
NTILES = 32
CPT = 80
NCH = NTILES * CPT
E_PAD = NCH * CHUNK
NROW = 10112
RPT = NROW // 16
N_PAD = 10240

_SC_MESH = plsc.VectorSubcoreMesh(core_axis_name="c", subcore_axis_name="s")


@functools.partial(
    pl.kernel,
    out_type=jax.ShapeDtypeStruct((2, NROW, D), jnp.float32),
    mesh=_SC_MESH,
    scratch_types=[
        pltpu.VMEM((CPT, CHUNK), jnp.int32),
        pltpu.VMEM((CHUNK, D), jnp.float32),
        pltpu.VMEM_SHARED((NROW, D), jnp.float32),
    ],
)
def _deg_kernel(dst_hbm, out_hbm, dst_buf, ones_buf, acc):
    c = lax.axis_index("c")
    s = lax.axis_index("s")
    tile = c * 16 + s

    def zrow(i, _):
        for k in range(D // 16):
            ones_buf[i, pl.ds(k * 16, 16)] = jnp.zeros((16,), jnp.float32)
        return 0
    lax.fori_loop(0, CHUNK, zrow, 0)

    r0 = s * RPT
    for k in range(RPT // CHUNK):
        pltpu.sync_copy(ones_buf, acc.at[pl.ds(r0 + k * CHUNK, CHUNK)])
    rem = RPT % CHUNK
    if rem:
        pltpu.sync_copy(ones_buf.at[pl.ds(0, rem)],
                        acc.at[pl.ds(r0 + (RPT // CHUNK) * CHUNK, rem)])

    def orow(i, _):
        for k in range(D // 16):
            ones_buf[i, pl.ds(k * 16, 16)] = jnp.ones((16,), jnp.float32)
        return 0
    lax.fori_loop(0, CHUNK, orow, 0)

    pltpu.sync_copy(dst_hbm.at[pl.ds(tile * CPT, CPT)], dst_buf)
    plsc.subcore_barrier()

    def body(j, _):
        pltpu.sync_copy(ones_buf, acc.at[dst_buf.at[j]], add=True)
        return 0
    lax.fori_loop(0, CPT, body, 0)

    plsc.subcore_barrier()
    pltpu.sync_copy(acc.at[pl.ds(r0, RPT)], out_hbm.at[c, pl.ds(r0, RPT)])


@functools.partial(
    pl.kernel,
    out_type=jax.ShapeDtypeStruct((2, NROW, D), jnp.float32),
    mesh=_SC_MESH,
    scratch_types=[
        pltpu.VMEM((CPT, CHUNK), jnp.int32),
        pltpu.VMEM((CPT, CHUNK), jnp.int32),
        pltpu.VMEM((CHUNK, D), jnp.float32),
        pltpu.VMEM_SHARED((NROW, D), jnp.float32),
        pltpu.SemaphoreType.DMA,
    ],
)
def _scatter_kernel(y_hbm, src_hbm, dst_hbm, out_hbm,
                    src_buf, dst_buf, rows, acc, sem):
    c = lax.axis_index("c")
    s = lax.axis_index("s")
    tile = c * 16 + s

    def zrow(i, _):
        for k in range(D // 16):
            rows[i, pl.ds(k * 16, 16)] = jnp.zeros((16,), jnp.float32)
        return 0
    lax.fori_loop(0, CHUNK, zrow, 0)

    r0 = s * RPT
    for k in range(RPT // CHUNK):
        pltpu.sync_copy(rows, acc.at[pl.ds(r0 + k * CHUNK, CHUNK)])
    rem = RPT % CHUNK
    if rem:
        pltpu.sync_copy(rows.at[pl.ds(0, rem)],
                        acc.at[pl.ds(r0 + (RPT // CHUNK) * CHUNK, rem)])

    pltpu.sync_copy(src_hbm.at[pl.ds(tile * CPT, CPT)], src_buf)
    pltpu.sync_copy(dst_hbm.at[pl.ds(tile * CPT, CPT)], dst_buf)
    plsc.subcore_barrier()

    def body(j, _):
        pltpu.async_copy(y_hbm.at[src_buf.at[j]], rows, sem).wait()
        pltpu.sync_copy(rows, acc.at[dst_buf.at[j]], add=True)
        return 0
    lax.fori_loop(0, CPT, body, 0)

    plsc.subcore_barrier()
    pltpu.sync_copy(acc.at[pl.ds(r0, RPT)], out_hbm.at[c, pl.ds(r0, RPT)])


@functools.partial(
    pl.kernel,
    out_type=(jax.ShapeDtypeStruct((NTILES * 8, D), jnp.float32),
              jax.ShapeDtypeStruct((NTILES * 8, D), jnp.float32)),
    mesh=_SC_MESH,
    scratch_types=[
        pltpu.VMEM((80,), jnp.float32),
        pltpu.VMEM((CHUNK, D), jnp.float32),
        pltpu.VMEM((CHUNK, D), jnp.float32),
        pltpu.VMEM((8, D), jnp.float32),
        pltpu.VMEM((8, D), jnp.float32),
    ],
)
def _pool_kernel(h2_hbm, hw_hbm, offs_hbm, wsum_hbm, wmax_hbm,
                 offs_buf, bufa, bufb, srows, mrows):
    c = lax.axis_index("c")
    s = lax.axis_index("s")
    wid = c * 16 + s
    pltpu.sync_copy(offs_hbm, offs_buf)
    nv = D // 16
    for lg in range(2):
        g = wid * 2 + lg
        ovec = offs_buf[pl.ds(g, 16)]
        start = ovec[0].astype(jnp.int32)
        end = ovec[1].astype(jnp.int32)
        start_al = pl.multiple_of((start // 8) * 8, 8)
        shift = start - start_al
        count = end - start_al
        nchunks = (count + CHUNK - 1) // CHUNK

        def chunk_body(i, carry):
            accs, accm = carry
            off = pl.multiple_of(start_al + i * CHUNK, 8)
            pltpu.sync_copy(h2_hbm.at[pl.ds(off, CHUNK)], bufa)
            pltpu.sync_copy(hw_hbm.at[pl.ds(off, CHUNK)], bufb)
            lo = jnp.where(i == 0, shift, 0)
            hi = jnp.minimum(count - i * CHUNK, CHUNK)

            def row_body(j, c2):
                a_s, a_m = c2
                new_s = tuple(a_s[k] + bufb[j, pl.ds(k * 16, 16)]
                              for k in range(nv))
                new_m = tuple(jnp.maximum(a_m[k], bufa[j, pl.ds(k * 16, 16)])
                              for k in range(nv))
                return (new_s, new_m)
            return lax.fori_loop(lo, hi, row_body, (accs, accm))

        init = (tuple(jnp.zeros((16,), jnp.float32) for _ in range(nv)),
                tuple(jnp.full((16,), -jnp.inf, jnp.float32) for _ in range(nv)))
        accs, accm = lax.fori_loop(0, nchunks, chunk_body, init)
        for k in range(nv):
            srows[lg, pl.ds(k * 16, 16)] = accs[k]
            mrows[lg, pl.ds(k * 16, 16)] = accm[k]
    pltpu.sync_copy(srows, wsum_hbm.at[pl.ds(wid * 8, 8)])
    pltpu.sync_copy(mrows, wmax_hbm.at[pl.ds(wid * 8, 8)])


_BLK1 = 1000
_BLK3 = 512


def _k1_body(x_ref, d0_ref, d1_ref, w1_ref, xl_ref, y_ref, dinv_ref):
    deg = d0_ref[...] + d1_ref[...] + 1.0
    dinv = lax.rsqrt(deg)
    xl = jnp.dot(x_ref[...], w1_ref[...], preferred_element_type=jnp.float32)
    xl_ref[...] = xl
    y_ref[...] = xl * dinv
    dinv_ref[...] = dinv


def _k2_body(a0_ref, a1_ref, xl1_ref, dinv_ref, w2_ref, b1_ref,
             xl2_ref, y2_ref):
    dv = dinv_ref[...]
    h1 = dv * (a0_ref[...] + a1_ref[...]) + dv * dv * xl1_ref[...] + b1_ref[...]
    xl2 = jnp.dot(h1, w2_ref[...], preferred_element_type=jnp.float32)
    xl2_ref[...] = xl2
    y2_ref[...] = xl2 * dv


def _k3_body(a0_ref, a1_ref, xl2_ref, dinv_ref, b2_ref, wp_ref, bp_ref,
             batch_ref, h2_ref, hw_ref, offs_ref, cnt):
    i = pl.program_id(0)
    dv = dinv_ref[...]
    h2 = dv * (a0_ref[...] + a1_ref[...]) + dv * dv * xl2_ref[...] + b2_ref[...]
    w = jax.nn.sigmoid(
        jnp.dot(h2, wp_ref[...], preferred_element_type=jnp.float32)
        + bp_ref[...])
    h2_ref[...] = h2
    hw_ref[...] = h2 * w
    gids = lax.broadcasted_iota(jnp.int32, (1, G), 1)
    oh = (batch_ref[...] == gids).astype(jnp.float32)
    bc = jnp.sum(oh, axis=0, keepdims=True)

    @pl.when(i == 0)
    def _():
        cnt[...] = bc

    @pl.when(i > 0)
    def _():
        cnt[...] = cnt[...] + bc

    counts = cnt[...]
    ii = lax.broadcasted_iota(jnp.int32, (G, G), 0)
    jj = lax.broadcasted_iota(jnp.int32, (G, G), 1)
    tri = (ii < jj).astype(jnp.float32)
    offs = jnp.dot(counts, tri, preferred_element_type=jnp.float32)
    offs_ref[...] = jnp.concatenate(
        [offs, jnp.full((1, 16), float(N), jnp.float32)], axis=1)


def _k4_body(ws_ref, wm_ref, add_ref, wg1_ref, wg2_ref, bgo_ref,
             wf1_ref, wf2_ref, bf_ref, out_ref):
    gmat = (jnp.dot(ws_ref[...], wg1_ref[...], preferred_element_type=jnp.float32)
            + jnp.dot(wm_ref[...], wg2_ref[...], preferred_element_type=jnp.float32)
            + bgo_ref[...])
    out_ref[...] = (
        jnp.dot(gmat, wf1_ref[...], preferred_element_type=jnp.float32)
        + jnp.dot(add_ref[...], wf2_ref[...], preferred_element_type=jnp.float32)
        + bf_ref[...])


def _row_spec(blk, width):
    return pl.BlockSpec((blk, width), lambda i: (i, 0))


def _full_spec(shape):
    return pl.BlockSpec(shape, lambda *i: (0,) * len(shape))


def kernel(x, edge_index, batch, additional_input, W1, b1, W2, b2,
           w_pool, b_pool, W_go, b_go, W_f, b_f):
    src = edge_index[0]
    dst = edge_index[1]
    pad = E_PAD - E
    src_p = jnp.concatenate([src, jnp.zeros((pad,), jnp.int32)]).reshape(NCH, CHUNK)
    dst_p = jnp.concatenate([dst, jnp.full((pad,), N, jnp.int32)]).reshape(NCH, CHUNK)

    degp = _deg_kernel(dst_p)
    d0 = degp[0, :N, 0:1]
    d1 = degp[1, :N, 0:1]

    xl1, y1, dinv = pl.pallas_call(
        _k1_body,
        grid=(N // _BLK1,),
        in_specs=[_row_spec(_BLK1, D), _row_spec(_BLK1, 1), _row_spec(_BLK1, 1),
                  _full_spec((D, D))],
        out_specs=[_row_spec(_BLK1, D), _row_spec(_BLK1, D), _row_spec(_BLK1, 1)],
        out_shape=[jax.ShapeDtypeStruct((N, D), jnp.float32),
                   jax.ShapeDtypeStruct((N, D), jnp.float32),
                   jax.ShapeDtypeStruct((N, 1), jnp.float32)],
    )(x, d0, d1, W1)

    acc1 = _scatter_kernel(y1, src_p, dst_p)

    xl2, y2 = pl.pallas_call(
        _k2_body,
        grid=(N // _BLK1,),
        in_specs=[_row_spec(_BLK1, D), _row_spec(_BLK1, D), _row_spec(_BLK1, D),
                  _row_spec(_BLK1, 1), _full_spec((D, D)), _full_spec((1, D))],
        out_specs=[_row_spec(_BLK1, D), _row_spec(_BLK1, D)],
        out_shape=[jax.ShapeDtypeStruct((N, D), jnp.float32),
                   jax.ShapeDtypeStruct((N, D), jnp.float32)],
    )(acc1[0, :N, :], acc1[1, :N, :], xl1, dinv, W2, b1.reshape(1, D))

    acc2 = _scatter_kernel(y2, src_p, dst_p)

    rpad = N_PAD - N
    a0p = jnp.pad(acc2[0, :N, :], ((0, rpad), (0, 0)))
    a1p = jnp.pad(acc2[1, :N, :], ((0, rpad), (0, 0)))
    xl2p = jnp.pad(xl2, ((0, rpad), (0, 0)))
    dinvp = jnp.pad(dinv, ((0, rpad), (0, 0)))
    batchp = jnp.pad(batch.reshape(N, 1), ((0, rpad), (0, 0)),
                     constant_values=G + 63)

    h2p, hwp, offs = pl.pallas_call(
        _k3_body,
        grid=(N_PAD // _BLK3,),
        in_specs=[_row_spec(_BLK3, D), _row_spec(_BLK3, D), _row_spec(_BLK3, D),
                  _row_spec(_BLK3, 1), _full_spec((1, D)), _full_spec((D, 1)),
                  _full_spec((1, 1)), _row_spec(_BLK3, 1)],
        out_specs=[_row_spec(_BLK3, D), _row_spec(_BLK3, D), _full_spec((1, 80))],
        out_shape=[jax.ShapeDtypeStruct((N_PAD, D), jnp.float32),
                   jax.ShapeDtypeStruct((N_PAD, D), jnp.float32),
                   jax.ShapeDtypeStruct((1, 80), jnp.float32)],
        scratch_shapes=[pltpu.VMEM((1, G), jnp.float32)],
    )(a0p, a1p, xl2p, dinvp, b2.reshape(1, D), w_pool,
      b_pool.reshape(1, 1), batchp)

    wsum8, wmax8 = _pool_kernel(h2p, hwp, offs.reshape(80))
    wsum = wsum8.reshape(NTILES, 8, D)[:, :2].reshape(G, D)
    wmax = wmax8.reshape(NTILES, 8, D)[:, :2].reshape(G, D)

    out = pl.pallas_call(
        _k4_body,
        in_specs=[_full_spec((G, D)), _full_spec((G, D)), _full_spec((G, 16)),
                  _full_spec((D, 256)), _full_spec((D, 256)), _full_spec((1, 256)),
                  _full_spec((256, 1)), _full_spec((16, 1)), _full_spec((1, 1))],
        out_specs=_full_spec((G, 1)),
        out_shape=jax.ShapeDtypeStruct((G, 1), jnp.float32),
    )(wsum, wmax, additional_input, W_go[:D], W_go[D:], b_go.reshape(1, 256),
      W_f[:256], W_f[256:], b_f.reshape(1, 1))

    return out


# speedup vs baseline: 18.3155x; 2.2634x over previous
import functools

import jax
import jax.numpy as jnp
from jax import lax
from jax.experimental import pallas as pl
from jax.experimental.pallas import tpu as pltpu
from jax.experimental.pallas import tpu_sc as plsc

N = 10000
E = 320000
D = 128
G = 64
CHUNK = 128
NTILES = 32
CPT = 80
NCH = NTILES * CPT
E_PAD = NCH * CHUNK
NROW = 10112
RPT = NROW // 16
N_PAD = 10240

_SC_MESH = plsc.VectorSubcoreMesh(core_axis_name="c", subcore_axis_name="s")


@functools.partial(
    pl.kernel,
    out_type=jax.ShapeDtypeStruct((2, NROW, D), jnp.float32),
    mesh=_SC_MESH,
    scratch_types=[
        pltpu.VMEM((CPT, CHUNK), jnp.int32),
        pltpu.VMEM((CHUNK, D), jnp.float32),
        pltpu.VMEM_SHARED((NROW, D), jnp.float32),
    ],
)
def _deg_kernel(dst_hbm, out_hbm, dst_buf, ones_buf, acc):
    c = lax.axis_index("c")
    s = lax.axis_index("s")
    tile = c * 16 + s

    def zrow(i, _):
        for k in range(D // 16):
            ones_buf[i, pl.ds(k * 16, 16)] = jnp.zeros((16,), jnp.float32)
        return 0
    lax.fori_loop(0, CHUNK, zrow, 0)

    r0 = s * RPT
    for k in range(RPT // CHUNK):
        pltpu.sync_copy(ones_buf, acc.at[pl.ds(r0 + k * CHUNK, CHUNK)])
    rem = RPT % CHUNK
    if rem:
        pltpu.sync_copy(ones_buf.at[pl.ds(0, rem)],
                        acc.at[pl.ds(r0 + (RPT // CHUNK) * CHUNK, rem)])

    def orow(i, _):
        for k in range(D // 16):
            ones_buf[i, pl.ds(k * 16, 16)] = jnp.ones((16,), jnp.float32)
        return 0
    lax.fori_loop(0, CHUNK, orow, 0)

    pltpu.sync_copy(dst_hbm.at[pl.ds(tile * CPT, CPT)], dst_buf)
    plsc.subcore_barrier()

    def body(j, _):
        pltpu.sync_copy(ones_buf, acc.at[dst_buf.at[j]], add=True)
        return 0
    lax.fori_loop(0, CPT, body, 0)

    plsc.subcore_barrier()
    pltpu.sync_copy(acc.at[pl.ds(r0, RPT)], out_hbm.at[c, pl.ds(r0, RPT)])


@functools.partial(
    pl.kernel,
    out_type=jax.ShapeDtypeStruct((2, NROW, D), jnp.float32),
    mesh=_SC_MESH,
    scratch_types=[
        pltpu.VMEM((CPT, CHUNK), jnp.int32),
        pltpu.VMEM((CPT, CHUNK), jnp.int32),
        pltpu.VMEM((CHUNK, D), jnp.float32),
        pltpu.VMEM_SHARED((NROW, D), jnp.float32),
        pltpu.SemaphoreType.DMA,
    ],
)
def _scatter_kernel(y_hbm, src_hbm, dst_hbm, out_hbm,
                    src_buf, dst_buf, rows, acc, sem):
    c = lax.axis_index("c")
    s = lax.axis_index("s")
    tile = c * 16 + s

    def zrow(i, _):
        for k in range(D // 16):
            rows[i, pl.ds(k * 16, 16)] = jnp.zeros((16,), jnp.float32)
        return 0
    lax.fori_loop(0, CHUNK, zrow, 0)

    r0 = s * RPT
    for k in range(RPT // CHUNK):
        pltpu.sync_copy(rows, acc.at[pl.ds(r0 + k * CHUNK, CHUNK)])
    rem = RPT % CHUNK
    if rem:
        pltpu.sync_copy(rows.at[pl.ds(0, rem)],
                        acc.at[pl.ds(r0 + (RPT // CHUNK) * CHUNK, rem)])

    pltpu.sync_copy(src_hbm.at[pl.ds(tile * CPT, CPT)], src_buf)
    pltpu.sync_copy(dst_hbm.at[pl.ds(tile * CPT, CPT)], dst_buf)
    plsc.subcore_barrier()

    def body(j, _):
        pltpu.async_copy(y_hbm.at[src_buf.at[j]], rows, sem).wait()
        pltpu.sync_copy(rows, acc.at[dst_buf.at[j]], add=True)
        return 0
    lax.fori_loop(0, CPT, body, 0)

    plsc.subcore_barrier()
    pltpu.sync_copy(acc.at[pl.ds(r0, RPT)], out_hbm.at[c, pl.ds(r0, RPT)])


@functools.partial(
    pl.kernel,
    out_type=(jax.ShapeDtypeStruct((NTILES * 8, D), jnp.float32),
              jax.ShapeDtypeStruct((NTILES * 8, D), jnp.float32)),
    mesh=_SC_MESH,
    scratch_types=[
        pltpu.VMEM((80,), jnp.float32),
        pltpu.VMEM((CHUNK, D), jnp.float32),
        pltpu.VMEM((CHUNK, D), jnp.float32),
        pltpu.VMEM((8, D), jnp.float32),
        pltpu.VMEM((8, D), jnp.float32),
    ],
)
def _pool_kernel(h2_hbm, hw_hbm, offs_hbm, wsum_hbm, wmax_hbm,
                 offs_buf, bufa, bufb, srows, mrows):
    c = lax.axis_index("c")
    s = lax.axis_index("s")
    wid = c * 16 + s
    pltpu.sync_copy(offs_hbm, offs_buf)
    nv = D // 16
    for lg in range(2):
        g = wid * 2 + lg
        ovec = offs_buf[pl.ds(g, 16)]
        start = ovec[0].astype(jnp.int32)
        end = ovec[1].astype(jnp.int32)
        start_al = pl.multiple_of((start // 8) * 8, 8)
        shift = start - start_al
        count = end - start_al
        nchunks = (count + CHUNK - 1) // CHUNK

        def chunk_body(i, carry):
            accs, accm = carry
            off = pl.multiple_of(start_al + i * CHUNK, 8)
            pltpu.sync_copy(h2_hbm.at[pl.ds(off, CHUNK)], bufa)
            pltpu.sync_copy(hw_hbm.at[pl.ds(off, CHUNK)], bufb)
            lo = jnp.where(i == 0, shift, 0)
            hi = jnp.minimum(count - i * CHUNK, CHUNK)

            def row_body(j, c2):
                a_s, a_m = c2
                new_s = tuple(a_s[k] + bufb[j, pl.ds(k * 16, 16)]
                              for k in range(nv))
                new_m = tuple(jnp.maximum(a_m[k], bufa[j, pl.ds(k * 16, 16)])
                              for k in range(nv))
                return (new_s, new_m)
            return lax.fori_loop(lo, hi, row_body, (accs, accm))

        init = (tuple(jnp.zeros((16,), jnp.float32) for _ in range(nv)),
                tuple(jnp.full((16,), -jnp.inf, jnp.float32) for _ in range(nv)))
        accs, accm = lax.fori_loop(0, nchunks, chunk_body, init)
        for k in range(nv):
            srows[lg, pl.ds(k * 16, 16)] = accs[k]
            mrows[lg, pl.ds(k * 16, 16)] = accm[k]
    pltpu.sync_copy(srows, wsum_hbm.at[pl.ds(wid * 8, 8)])
    pltpu.sync_copy(mrows, wmax_hbm.at[pl.ds(wid * 8, 8)])


_BLK1 = 1000
_BLK3 = 512


def _k1_body(x_ref, d0_ref, d1_ref, w1_ref, xl_ref, y_ref, dinv_ref):
    deg = d0_ref[...] + d1_ref[...] + 1.0
    dinv = lax.rsqrt(deg)
    xl = jnp.dot(x_ref[...], w1_ref[...], preferred_element_type=jnp.float32)
    xl_ref[...] = xl
    y_ref[...] = xl * dinv
    dinv_ref[...] = dinv


def _k2_body(a0_ref, a1_ref, xl1_ref, dinv_ref, w2_ref, b1_ref,
             xl2_ref, y2_ref):
    dv = dinv_ref[...]
    h1 = dv * (a0_ref[...] + a1_ref[...]) + dv * dv * xl1_ref[...] + b1_ref[...]
    xl2 = jnp.dot(h1, w2_ref[...], preferred_element_type=jnp.float32)
    xl2_ref[...] = xl2
    y2_ref[...] = xl2 * dv


def _k3_body(a0_ref, a1_ref, xl2_ref, dinv_ref, b2_ref, wp_ref, bp_ref,
             batch_ref, h2_ref, hw_ref, offs_ref, cnt):
    i = pl.program_id(0)
    dv = dinv_ref[...]
    h2 = dv * (a0_ref[...] + a1_ref[...]) + dv * dv * xl2_ref[...] + b2_ref[...]
    w = jax.nn.sigmoid(
        jnp.dot(h2, wp_ref[...], preferred_element_type=jnp.float32)
        + bp_ref[...])
    h2_ref[...] = h2
    hw_ref[...] = h2 * w
    gids = lax.broadcasted_iota(jnp.int32, (1, G), 1)
    oh = (batch_ref[...] == gids).astype(jnp.float32)
    bc = jnp.sum(oh, axis=0, keepdims=True)

    @pl.when(i == 0)
    def _():
        cnt[...] = bc

    @pl.when(i > 0)
    def _():
        cnt[...] = cnt[...] + bc

    counts = cnt[...]
    ii = lax.broadcasted_iota(jnp.int32, (G, G), 0)
    jj = lax.broadcasted_iota(jnp.int32, (G, G), 1)
    tri = (ii < jj).astype(jnp.float32)
    offs = jnp.dot(counts, tri, preferred_element_type=jnp.float32)
    offs_ref[...] = jnp.concatenate(
        [offs, jnp.full((1, 16), float(N), jnp.float32)], axis=1)


def _k4_body(ws_ref, wm_ref, add_ref, wg1_ref, wg2_ref, bgo_ref,
             wf1_ref, wf2_ref, bf_ref, out_ref):
    gmat = (jnp.dot(ws_ref[...], wg1_ref[...], preferred_element_type=jnp.float32)
            + jnp.dot(wm_ref[...], wg2_ref[...], preferred_element_type=jnp.float32)
            + bgo_ref[...])
    out_ref[...] = (
        jnp.dot(gmat, wf1_ref[...], preferred_element_type=jnp.float32)
        + jnp.dot(add_ref[...], wf2_ref[...], preferred_element_type=jnp.float32)
        + bf_ref[...])


def _row_spec(blk, width):
    return pl.BlockSpec((blk, width), lambda i: (i, 0))


def _full_spec(shape):
    return pl.BlockSpec(shape, lambda *i: (0,) * len(shape))


def kernel(x, edge_index, batch, additional_input, W1, b1, W2, b2,
           w_pool, b_pool, W_go, b_go, W_f, b_f):
    src = edge_index[0]
    dst = edge_index[1]
    pad = E_PAD - E
    pad_idx = jnp.arange(pad, dtype=jnp.int32)
    src_p = jnp.concatenate([src, pad_idx % N]).reshape(NCH, CHUNK)
    dst_p = jnp.concatenate([dst, N + pad_idx % (NROW - N)]).reshape(NCH, CHUNK)

    degp = _deg_kernel(dst_p)
    d0 = degp[0, :N, 0:1]
    d1 = degp[1, :N, 0:1]

    xl1, y1, dinv = pl.pallas_call(
        _k1_body,
        grid=(N // _BLK1,),
        in_specs=[_row_spec(_BLK1, D), _row_spec(_BLK1, 1), _row_spec(_BLK1, 1),
                  _full_spec((D, D))],
        out_specs=[_row_spec(_BLK1, D), _row_spec(_BLK1, D), _row_spec(_BLK1, 1)],
        out_shape=[jax.ShapeDtypeStruct((N, D), jnp.float32),
                   jax.ShapeDtypeStruct((N, D), jnp.float32),
                   jax.ShapeDtypeStruct((N, 1), jnp.float32)],
    )(x, d0, d1, W1)

    acc1 = _scatter_kernel(y1, src_p, dst_p)

    xl2, y2 = pl.pallas_call(
        _k2_body,
        grid=(N // _BLK1,),
        in_specs=[_row_spec(_BLK1, D), _row_spec(_BLK1, D), _row_spec(_BLK1, D),
                  _row_spec(_BLK1, 1), _full_spec((D, D)), _full_spec((1, D))],
        out_specs=[_row_spec(_BLK1, D), _row_spec(_BLK1, D)],
        out_shape=[jax.ShapeDtypeStruct((N, D), jnp.float32),
                   jax.ShapeDtypeStruct((N, D), jnp.float32)],
    )(acc1[0, :N, :], acc1[1, :N, :], xl1, dinv, W2, b1.reshape(1, D))

    acc2 = _scatter_kernel(y2, src_p, dst_p)

    rpad = N_PAD - N
    a0p = jnp.pad(acc2[0, :N, :], ((0, rpad), (0, 0)))
    a1p = jnp.pad(acc2[1, :N, :], ((0, rpad), (0, 0)))
    xl2p = jnp.pad(xl2, ((0, rpad), (0, 0)))
    dinvp = jnp.pad(dinv, ((0, rpad), (0, 0)))
    batchp = jnp.pad(batch.reshape(N, 1), ((0, rpad), (0, 0)),
                     constant_values=G + 63)

    h2p, hwp, offs = pl.pallas_call(
        _k3_body,
        grid=(N_PAD // _BLK3,),
        in_specs=[_row_spec(_BLK3, D), _row_spec(_BLK3, D), _row_spec(_BLK3, D),
                  _row_spec(_BLK3, 1), _full_spec((1, D)), _full_spec((D, 1)),
                  _full_spec((1, 1)), _row_spec(_BLK3, 1)],
        out_specs=[_row_spec(_BLK3, D), _row_spec(_BLK3, D), _full_spec((1, 80))],
        out_shape=[jax.ShapeDtypeStruct((N_PAD, D), jnp.float32),
                   jax.ShapeDtypeStruct((N_PAD, D), jnp.float32),
                   jax.ShapeDtypeStruct((1, 80), jnp.float32)],
        scratch_shapes=[pltpu.VMEM((1, G), jnp.float32)],
    )(a0p, a1p, xl2p, dinvp, b2.reshape(1, D), w_pool,
      b_pool.reshape(1, 1), batchp)

    wsum8, wmax8 = _pool_kernel(h2p, hwp, offs.reshape(80))
    wsum = wsum8.reshape(NTILES, 8, D)[:, :2].reshape(G, D)
    wmax = wmax8.reshape(NTILES, 8, D)[:, :2].reshape(G, D)

    out = pl.pallas_call(
        _k4_body,
        in_specs=[_full_spec((G, D)), _full_spec((G, D)), _full_spec((G, 16)),
                  _full_spec((D, 256)), _full_spec((D, 256)), _full_spec((1, 256)),
                  _full_spec((256, 1)), _full_spec((16, 1)), _full_spec((1, 1))],
        out_specs=_full_spec((G, 1)),
        out_shape=jax.ShapeDtypeStruct((G, 1), jnp.float32),
    )(wsum, wmax, additional_input, W_go[:D], W_go[D:], b_go.reshape(1, 256),
      W_f[:256], W_f[256:], b_f.reshape(1, 1))

    return out


# speedup vs baseline: 24.0862x; 1.3151x over previous
import functools

import jax
import jax.numpy as jnp
from jax import lax
from jax.experimental import pallas as pl
from jax.experimental.pallas import tpu as pltpu
from jax.experimental.pallas import tpu_sc as plsc

N = 10000
E = 320000
D = 128
G = 64
CHUNK = 128
NTILES = 32
CPT = 80
NCH = NTILES * CPT
E_PAD = NCH * CHUNK
NROW = 10112
RPT = NROW // 16
N_PAD = 10240

_SC_MESH = plsc.VectorSubcoreMesh(core_axis_name="c", subcore_axis_name="s")


@functools.partial(
    pl.kernel,
    out_type=jax.ShapeDtypeStruct((2, NROW, D), jnp.float32),
    mesh=_SC_MESH,
    scratch_types=[
        pltpu.VMEM((CPT, CHUNK), jnp.int32),
        pltpu.VMEM((CHUNK, D), jnp.float32),
        pltpu.VMEM_SHARED((NROW, D), jnp.float32),
    ],
)
def _deg_kernel(dst_hbm, out_hbm, dst_buf, ones_buf, acc):
    c = lax.axis_index("c")
    s = lax.axis_index("s")
    tile = c * 16 + s

    def zrow(i, _):
        for k in range(D // 16):
            ones_buf[i, pl.ds(k * 16, 16)] = jnp.zeros((16,), jnp.float32)
        return 0
    lax.fori_loop(0, CHUNK, zrow, 0)

    r0 = s * RPT
    for k in range(RPT // CHUNK):
        pltpu.sync_copy(ones_buf, acc.at[pl.ds(r0 + k * CHUNK, CHUNK)])
    rem = RPT % CHUNK
    if rem:
        pltpu.sync_copy(ones_buf.at[pl.ds(0, rem)],
                        acc.at[pl.ds(r0 + (RPT // CHUNK) * CHUNK, rem)])

    def orow(i, _):
        for k in range(D // 16):
            ones_buf[i, pl.ds(k * 16, 16)] = jnp.ones((16,), jnp.float32)
        return 0
    lax.fori_loop(0, CHUNK, orow, 0)

    pltpu.sync_copy(dst_hbm.at[pl.ds(tile * CPT, CPT)], dst_buf)
    plsc.subcore_barrier()

    def body(j, _):
        pltpu.sync_copy(ones_buf, acc.at[dst_buf.at[j]], add=True)
        return 0
    lax.fori_loop(0, CPT, body, 0)

    plsc.subcore_barrier()
    pltpu.sync_copy(acc.at[pl.ds(r0, RPT)], out_hbm.at[c, pl.ds(r0, RPT)])


NB = 40


@functools.partial(
    pl.kernel,
    out_type=jax.ShapeDtypeStruct((2, NROW, D), jnp.float32),
    mesh=_SC_MESH,
    scratch_types=[
        pltpu.VMEM((NB, CHUNK), jnp.int32),
        pltpu.VMEM((NB, CHUNK), jnp.int32),
        pltpu.VMEM((CHUNK, D), jnp.float32),
        pltpu.VMEM((CHUNK, D), jnp.float32),
        pltpu.VMEM_SHARED((NROW, D), jnp.float32),
        pltpu.SemaphoreType.DMA,
        pltpu.SemaphoreType.DMA,
    ],
)
def _scatter_kernel(y_hbm, src_hbm, dst_hbm, out_hbm,
                    src_buf, dst_buf, rows0, rows1, acc, sem0, sem1):
    c = lax.axis_index("c")
    s = lax.axis_index("s")
    tile = c * 16 + s
    bufs = (rows0, rows1)
    sems = (sem0, sem1)

    def zrow(i, _):
        for k in range(D // 16):
            rows0[i, pl.ds(k * 16, 16)] = jnp.zeros((16,), jnp.float32)
        return 0
    lax.fori_loop(0, CHUNK, zrow, 0)

    r0 = s * RPT
    for k in range(RPT // CHUNK):
        pltpu.sync_copy(rows0, acc.at[pl.ds(r0 + k * CHUNK, CHUNK)])
    rem = RPT % CHUNK
    if rem:
        pltpu.sync_copy(rows0.at[pl.ds(0, rem)],
                        acc.at[pl.ds(r0 + (RPT // CHUNK) * CHUNK, rem)])
    plsc.subcore_barrier()

    def wait_gather(b):
        pltpu.make_async_copy(y_hbm.at[pl.ds(0, CHUNK)], bufs[b], sems[b]).wait()

    for blk in range(CPT // NB):
        base = tile * CPT + blk * NB
        pltpu.sync_copy(src_hbm.at[pl.ds(base, NB)], src_buf)
        pltpu.sync_copy(dst_hbm.at[pl.ds(base, NB)], dst_buf)
        for b in range(2):
            pltpu.async_copy(y_hbm.at[src_buf.at[b]], bufs[b], sems[b])

        def pair_body(jj, _):
            for b in range(2):
                j = jj * 2 + b
                wait_gather(b)
                pltpu.sync_copy(bufs[b], acc.at[dst_buf.at[j]], add=True)

                @pl.when(j + 2 < NB)
                def _():
                    pltpu.async_copy(y_hbm.at[src_buf.at[j + 2]],
                                     bufs[b], sems[b])
            return 0
        lax.fori_loop(0, NB // 2, pair_body, 0)

    plsc.subcore_barrier()
    pltpu.sync_copy(acc.at[pl.ds(r0, RPT)], out_hbm.at[c, pl.ds(r0, RPT)])


@functools.partial(
    pl.kernel,
    out_type=(jax.ShapeDtypeStruct((NTILES * 8, D), jnp.float32),
              jax.ShapeDtypeStruct((NTILES * 8, D), jnp.float32)),
    mesh=_SC_MESH,
    scratch_types=[
        pltpu.VMEM((80,), jnp.float32),
        pltpu.VMEM((CHUNK, D), jnp.float32),
        pltpu.VMEM((CHUNK, D), jnp.float32),
        pltpu.VMEM((8, D), jnp.float32),
        pltpu.VMEM((8, D), jnp.float32),
    ],
)
def _pool_kernel(h2_hbm, hw_hbm, offs_hbm, wsum_hbm, wmax_hbm,
                 offs_buf, bufa, bufb, srows, mrows):
    c = lax.axis_index("c")
    s = lax.axis_index("s")
    wid = c * 16 + s
    pltpu.sync_copy(offs_hbm, offs_buf)
    nv = D // 16
    for lg in range(2):
        g = wid * 2 + lg
        ovec = offs_buf[pl.ds(g, 16)]
        start = ovec[0].astype(jnp.int32)
        end = ovec[1].astype(jnp.int32)
        start_al = pl.multiple_of((start // 8) * 8, 8)
        shift = start - start_al
        count = end - start_al
        nchunks = (count + CHUNK - 1) // CHUNK

        def chunk_body(i, carry):
            accs, accm = carry
            off = pl.multiple_of(start_al + i * CHUNK, 8)
            pltpu.sync_copy(h2_hbm.at[pl.ds(off, CHUNK)], bufa)
            pltpu.sync_copy(hw_hbm.at[pl.ds(off, CHUNK)], bufb)
            lo = jnp.where(i == 0, shift, 0)
            hi = jnp.minimum(count - i * CHUNK, CHUNK)

            def row_body(j, c2):
                a_s, a_m = c2
                new_s = tuple(a_s[k] + bufb[j, pl.ds(k * 16, 16)]
                              for k in range(nv))
                new_m = tuple(jnp.maximum(a_m[k], bufa[j, pl.ds(k * 16, 16)])
                              for k in range(nv))
                return (new_s, new_m)
            return lax.fori_loop(lo, hi, row_body, (accs, accm))

        init = (tuple(jnp.zeros((16,), jnp.float32) for _ in range(nv)),
                tuple(jnp.full((16,), -jnp.inf, jnp.float32) for _ in range(nv)))
        accs, accm = lax.fori_loop(0, nchunks, chunk_body, init)
        for k in range(nv):
            srows[lg, pl.ds(k * 16, 16)] = accs[k]
            mrows[lg, pl.ds(k * 16, 16)] = accm[k]
    pltpu.sync_copy(srows, wsum_hbm.at[pl.ds(wid * 8, 8)])
    pltpu.sync_copy(mrows, wmax_hbm.at[pl.ds(wid * 8, 8)])


_BLK1 = 1000
_BLK3 = 512


def _k1_body(x_ref, d0_ref, d1_ref, w1_ref, xl_ref, y_ref, dinv_ref):
    deg = d0_ref[...] + d1_ref[...] + 1.0
    dinv = lax.rsqrt(deg)
    xl = jnp.dot(x_ref[...], w1_ref[...], preferred_element_type=jnp.float32)
    xl_ref[...] = xl
    y_ref[...] = xl * dinv
    dinv_ref[...] = dinv


def _k2_body(a0_ref, a1_ref, xl1_ref, dinv_ref, w2_ref, b1_ref,
             xl2_ref, y2_ref):
    dv = dinv_ref[...]
    h1 = dv * (a0_ref[...] + a1_ref[...]) + dv * dv * xl1_ref[...] + b1_ref[...]
    xl2 = jnp.dot(h1, w2_ref[...], preferred_element_type=jnp.float32)
    xl2_ref[...] = xl2
    y2_ref[...] = xl2 * dv


def _k3_body(a0_ref, a1_ref, xl2_ref, dinv_ref, b2_ref, wp_ref, bp_ref,
             batch_ref, h2_ref, hw_ref, offs_ref, cnt):
    i = pl.program_id(0)
    dv = dinv_ref[...]
    h2 = dv * (a0_ref[...] + a1_ref[...]) + dv * dv * xl2_ref[...] + b2_ref[...]
    w = jax.nn.sigmoid(
        jnp.dot(h2, wp_ref[...], preferred_element_type=jnp.float32)
        + bp_ref[...])
    h2_ref[...] = h2
    hw_ref[...] = h2 * w
    gids = lax.broadcasted_iota(jnp.int32, (1, G), 1)
    oh = (batch_ref[...] == gids).astype(jnp.float32)
    bc = jnp.sum(oh, axis=0, keepdims=True)

    @pl.when(i == 0)
    def _():
        cnt[...] = bc

    @pl.when(i > 0)
    def _():
        cnt[...] = cnt[...] + bc

    counts = cnt[...]
    ii = lax.broadcasted_iota(jnp.int32, (G, G), 0)
    jj = lax.broadcasted_iota(jnp.int32, (G, G), 1)
    tri = (ii < jj).astype(jnp.float32)
    offs = jnp.dot(counts, tri, preferred_element_type=jnp.float32)
    offs_ref[...] = jnp.concatenate(
        [offs, jnp.full((1, 16), float(N), jnp.float32)], axis=1)


def _k4_body(ws_ref, wm_ref, add_ref, wg1_ref, wg2_ref, bgo_ref,
             wf1_ref, wf2_ref, bf_ref, out_ref):
    gmat = (jnp.dot(ws_ref[...], wg1_ref[...], preferred_element_type=jnp.float32)
            + jnp.dot(wm_ref[...], wg2_ref[...], preferred_element_type=jnp.float32)
            + bgo_ref[...])
    out_ref[...] = (
        jnp.dot(gmat, wf1_ref[...], preferred_element_type=jnp.float32)
        + jnp.dot(add_ref[...], wf2_ref[...], preferred_element_type=jnp.float32)
        + bf_ref[...])


def _row_spec(blk, width):
    return pl.BlockSpec((blk, width), lambda i: (i, 0))


def _full_spec(shape):
    return pl.BlockSpec(shape, lambda *i: (0,) * len(shape))


def kernel(x, edge_index, batch, additional_input, W1, b1, W2, b2,
           w_pool, b_pool, W_go, b_go, W_f, b_f):
    src = edge_index[0]
    dst = edge_index[1]
    pad = E_PAD - E
    pad_idx = jnp.arange(pad, dtype=jnp.int32)
    src_p = jnp.concatenate([src, pad_idx % N]).reshape(NCH, CHUNK)
    dst_p = jnp.concatenate([dst, N + pad_idx % (NROW - N)]).reshape(NCH, CHUNK)

    degp = _deg_kernel(dst_p)
    d0 = degp[0, :N, 0:1]
    d1 = degp[1, :N, 0:1]

    xl1, y1, dinv = pl.pallas_call(
        _k1_body,
        grid=(N // _BLK1,),
        in_specs=[_row_spec(_BLK1, D), _row_spec(_BLK1, 1), _row_spec(_BLK1, 1),
                  _full_spec((D, D))],
        out_specs=[_row_spec(_BLK1, D), _row_spec(_BLK1, D), _row_spec(_BLK1, 1)],
        out_shape=[jax.ShapeDtypeStruct((N, D), jnp.float32),
                   jax.ShapeDtypeStruct((N, D), jnp.float32),
                   jax.ShapeDtypeStruct((N, 1), jnp.float32)],
    )(x, d0, d1, W1)

    acc1 = _scatter_kernel(y1, src_p, dst_p)

    xl2, y2 = pl.pallas_call(
        _k2_body,
        grid=(N // _BLK1,),
        in_specs=[_row_spec(_BLK1, D), _row_spec(_BLK1, D), _row_spec(_BLK1, D),
                  _row_spec(_BLK1, 1), _full_spec((D, D)), _full_spec((1, D))],
        out_specs=[_row_spec(_BLK1, D), _row_spec(_BLK1, D)],
        out_shape=[jax.ShapeDtypeStruct((N, D), jnp.float32),
                   jax.ShapeDtypeStruct((N, D), jnp.float32)],
    )(acc1[0, :N, :], acc1[1, :N, :], xl1, dinv, W2, b1.reshape(1, D))

    acc2 = _scatter_kernel(y2, src_p, dst_p)

    rpad = N_PAD - N
    a0p = jnp.pad(acc2[0, :N, :], ((0, rpad), (0, 0)))
    a1p = jnp.pad(acc2[1, :N, :], ((0, rpad), (0, 0)))
    xl2p = jnp.pad(xl2, ((0, rpad), (0, 0)))
    dinvp = jnp.pad(dinv, ((0, rpad), (0, 0)))
    batchp = jnp.pad(batch.reshape(N, 1), ((0, rpad), (0, 0)),
                     constant_values=G + 63)

    h2p, hwp, offs = pl.pallas_call(
        _k3_body,
        grid=(N_PAD // _BLK3,),
        in_specs=[_row_spec(_BLK3, D), _row_spec(_BLK3, D), _row_spec(_BLK3, D),
                  _row_spec(_BLK3, 1), _full_spec((1, D)), _full_spec((D, 1)),
                  _full_spec((1, 1)), _row_spec(_BLK3, 1)],
        out_specs=[_row_spec(_BLK3, D), _row_spec(_BLK3, D), _full_spec((1, 80))],
        out_shape=[jax.ShapeDtypeStruct((N_PAD, D), jnp.float32),
                   jax.ShapeDtypeStruct((N_PAD, D), jnp.float32),
                   jax.ShapeDtypeStruct((1, 80), jnp.float32)],
        scratch_shapes=[pltpu.VMEM((1, G), jnp.float32)],
    )(a0p, a1p, xl2p, dinvp, b2.reshape(1, D), w_pool,
      b_pool.reshape(1, 1), batchp)

    wsum8, wmax8 = _pool_kernel(h2p, hwp, offs.reshape(80))
    wsum = wsum8.reshape(NTILES, 8, D)[:, :2].reshape(G, D)
    wmax = wmax8.reshape(NTILES, 8, D)[:, :2].reshape(G, D)

    out = pl.pallas_call(
        _k4_body,
        in_specs=[_full_spec((G, D)), _full_spec((G, D)), _full_spec((G, 16)),
                  _full_spec((D, 256)), _full_spec((D, 256)), _full_spec((1, 256)),
                  _full_spec((256, 1)), _full_spec((16, 1)), _full_spec((1, 1))],
        out_specs=_full_spec((G, 1)),
        out_shape=jax.ShapeDtypeStruct((G, 1), jnp.float32),
    )(wsum, wmax, additional_input, W_go[:D], W_go[D:], b_go.reshape(1, 256),
      W_f[:256], W_f[256:], b_f.reshape(1, 1))

    return out


# speedup vs baseline: 24.7589x; 1.0279x over previous
import functools

import jax
import jax.numpy as jnp
from jax import lax
from jax.experimental import pallas as pl
from jax.experimental.pallas import tpu as pltpu
from jax.experimental.pallas import tpu_sc as plsc

N = 10000
E = 320000
D = 128
G = 64
CHUNK = 128
NTILES = 32
CPT = 80
NCH = NTILES * CPT
E_PAD = NCH * CHUNK
NROW = 10240
RPT = NROW // 16

_SC_MESH = plsc.VectorSubcoreMesh(core_axis_name="c", subcore_axis_name="s")


@functools.partial(
    pl.kernel,
    out_type=jax.ShapeDtypeStruct((2, NROW, D), jnp.float32),
    mesh=_SC_MESH,
    scratch_types=[
        pltpu.VMEM((CPT, CHUNK), jnp.int32),
        pltpu.VMEM((CHUNK, D), jnp.float32),
        pltpu.VMEM_SHARED((NROW, D), jnp.float32),
    ],
)
def _deg_kernel(dst_hbm, out_hbm, dst_buf, ones_buf, acc):
    c = lax.axis_index("c")
    s = lax.axis_index("s")
    tile = c * 16 + s

    def zrow(i, _):
        for k in range(D // 16):
            ones_buf[i, pl.ds(k * 16, 16)] = jnp.zeros((16,), jnp.float32)
        return 0
    lax.fori_loop(0, CHUNK, zrow, 0)

    r0 = s * RPT
    for k in range(RPT // CHUNK):
        pltpu.sync_copy(ones_buf, acc.at[pl.ds(r0 + k * CHUNK, CHUNK)])
    rem = RPT % CHUNK
    if rem:
        pltpu.sync_copy(ones_buf.at[pl.ds(0, rem)],
                        acc.at[pl.ds(r0 + (RPT // CHUNK) * CHUNK, rem)])

    def orow(i, _):
        for k in range(D // 16):
            ones_buf[i, pl.ds(k * 16, 16)] = jnp.ones((16,), jnp.float32)
        return 0
    lax.fori_loop(0, CHUNK, orow, 0)

    pltpu.sync_copy(dst_hbm.at[pl.ds(tile * CPT, CPT)], dst_buf)
    plsc.subcore_barrier()

    def body(j, _):
        pltpu.sync_copy(ones_buf, acc.at[dst_buf.at[j]], add=True)
        return 0
    lax.fori_loop(0, CPT, body, 0)

    plsc.subcore_barrier()
    pltpu.sync_copy(acc.at[pl.ds(r0, RPT)], out_hbm.at[c, pl.ds(r0, RPT)])


NB = 40


@functools.partial(
    pl.kernel,
    out_type=jax.ShapeDtypeStruct((2, NROW, D), jnp.float32),
    mesh=_SC_MESH,
    scratch_types=[
        pltpu.VMEM((NB, CHUNK), jnp.int32),
        pltpu.VMEM((NB, CHUNK), jnp.int32),
        pltpu.VMEM((CHUNK, D), jnp.float32),
        pltpu.VMEM((CHUNK, D), jnp.float32),
        pltpu.VMEM_SHARED((NROW, D), jnp.float32),
        pltpu.SemaphoreType.DMA,
        pltpu.SemaphoreType.DMA,
    ],
)
def _scatter_kernel(y_hbm, src_hbm, dst_hbm, out_hbm,
                    src_buf, dst_buf, rows0, rows1, acc, sem0, sem1):
    c = lax.axis_index("c")
    s = lax.axis_index("s")
    tile = c * 16 + s
    bufs = (rows0, rows1)
    sems = (sem0, sem1)

    def zrow(i, _):
        for k in range(D // 16):
            rows0[i, pl.ds(k * 16, 16)] = jnp.zeros((16,), jnp.float32)
        return 0
    lax.fori_loop(0, CHUNK, zrow, 0)

    r0 = s * RPT
    for k in range(RPT // CHUNK):
        pltpu.sync_copy(rows0, acc.at[pl.ds(r0 + k * CHUNK, CHUNK)])
    rem = RPT % CHUNK
    if rem:
        pltpu.sync_copy(rows0.at[pl.ds(0, rem)],
                        acc.at[pl.ds(r0 + (RPT // CHUNK) * CHUNK, rem)])
    plsc.subcore_barrier()

    def wait_gather(b):
        pltpu.make_async_copy(y_hbm.at[pl.ds(0, CHUNK)], bufs[b], sems[b]).wait()

    for blk in range(CPT // NB):
        base = tile * CPT + blk * NB
        pltpu.sync_copy(src_hbm.at[pl.ds(base, NB)], src_buf)
        pltpu.sync_copy(dst_hbm.at[pl.ds(base, NB)], dst_buf)
        for b in range(2):
            pltpu.async_copy(y_hbm.at[src_buf.at[b]], bufs[b], sems[b])

        def pair_body(jj, _):
            for b in range(2):
                j = jj * 2 + b
                wait_gather(b)
                pltpu.sync_copy(bufs[b], acc.at[dst_buf.at[j]], add=True)

                @pl.when(j + 2 < NB)
                def _():
                    pltpu.async_copy(y_hbm.at[src_buf.at[j + 2]],
                                     bufs[b], sems[b])
            return 0
        lax.fori_loop(0, NB // 2, pair_body, 0)

    plsc.subcore_barrier()
    pltpu.sync_copy(acc.at[pl.ds(r0, RPT)], out_hbm.at[c, pl.ds(r0, RPT)])


@functools.partial(
    pl.kernel,
    out_type=(jax.ShapeDtypeStruct((NTILES * 8, D), jnp.float32),
              jax.ShapeDtypeStruct((NTILES * 8, D), jnp.float32)),
    mesh=_SC_MESH,
    scratch_types=[
        pltpu.VMEM((80,), jnp.float32),
        pltpu.VMEM((CHUNK, D), jnp.float32),
        pltpu.VMEM((CHUNK, D), jnp.float32),
        pltpu.VMEM((8, D), jnp.float32),
        pltpu.VMEM((8, D), jnp.float32),
    ],
)
def _pool_kernel(h2_hbm, hw_hbm, offs_hbm, wsum_hbm, wmax_hbm,
                 offs_buf, bufa, bufb, srows, mrows):
    c = lax.axis_index("c")
    s = lax.axis_index("s")
    wid = c * 16 + s
    pltpu.sync_copy(offs_hbm, offs_buf)
    nv = D // 16
    for lg in range(2):
        g = wid * 2 + lg
        ovec = offs_buf[pl.ds(g, 16)]
        start = ovec[0].astype(jnp.int32)
        end = ovec[1].astype(jnp.int32)
        start_al = pl.multiple_of((start // 8) * 8, 8)
        shift = start - start_al
        count = end - start_al
        nchunks = (count + CHUNK - 1) // CHUNK

        def chunk_body(i, carry):
            accs, accm = carry
            off = pl.multiple_of(start_al + i * CHUNK, 8)
            pltpu.sync_copy(h2_hbm.at[pl.ds(off, CHUNK)], bufa)
            pltpu.sync_copy(hw_hbm.at[pl.ds(off, CHUNK)], bufb)
            lo = jnp.where(i == 0, shift, 0)
            hi = jnp.minimum(count - i * CHUNK, CHUNK)

            def row_body(j, c2):
                a_s, a_m = c2
                new_s = tuple(a_s[k] + bufb[j, pl.ds(k * 16, 16)]
                              for k in range(nv))
                new_m = tuple(jnp.maximum(a_m[k], bufa[j, pl.ds(k * 16, 16)])
                              for k in range(nv))
                return (new_s, new_m)
            return lax.fori_loop(lo, hi, row_body, (accs, accm))

        init = (tuple(jnp.zeros((16,), jnp.float32) for _ in range(nv)),
                tuple(jnp.full((16,), -jnp.inf, jnp.float32) for _ in range(nv)))
        accs, accm = lax.fori_loop(0, nchunks, chunk_body, init)
        for k in range(nv):
            srows[lg, pl.ds(k * 16, 16)] = accs[k]
            mrows[lg, pl.ds(k * 16, 16)] = accm[k]
    pltpu.sync_copy(srows, wsum_hbm.at[pl.ds(wid * 8, 8)])
    pltpu.sync_copy(mrows, wmax_hbm.at[pl.ds(wid * 8, 8)])


_BLK = 1024


def _k1a_body(x_ref, w1_ref, xl_ref):
    xl_ref[...] = jnp.dot(x_ref[...], w1_ref[...],
                          preferred_element_type=jnp.float32)


def _k1b_body(xl_ref, d0_ref, d1_ref, y_ref, dinv_ref):
    deg = d0_ref[...] + d1_ref[...] + 1.0
    dinv = lax.rsqrt(deg)
    y_ref[...] = xl_ref[...] * dinv
    dinv_ref[...] = dinv


def _k2_body(a0_ref, a1_ref, xl1_ref, dinv_ref, w2_ref, b1_ref,
             xl2_ref, y2_ref):
    dv = dinv_ref[...]
    h1 = dv * (a0_ref[...] + a1_ref[...]) + dv * dv * xl1_ref[...] + b1_ref[...]
    xl2 = jnp.dot(h1, w2_ref[...], preferred_element_type=jnp.float32)
    xl2_ref[...] = xl2
    y2_ref[...] = xl2 * dv


def _k3_body(a0_ref, a1_ref, xl2_ref, dinv_ref, b2_ref, wp_ref, bp_ref,
             batch_ref, h2_ref, hw_ref, offs_ref, cnt):
    i = pl.program_id(0)
    dv = dinv_ref[...]
    h2 = dv * (a0_ref[...] + a1_ref[...]) + dv * dv * xl2_ref[...] + b2_ref[...]
    w = jax.nn.sigmoid(
        jnp.dot(h2, wp_ref[...], preferred_element_type=jnp.float32)
        + bp_ref[...])
    h2_ref[...] = h2
    hw_ref[...] = h2 * w
    gids = lax.broadcasted_iota(jnp.int32, (1, G), 1)
    oh = (batch_ref[...] == gids).astype(jnp.float32)
    bc = jnp.sum(oh, axis=0, keepdims=True)

    @pl.when(i == 0)
    def _():
        cnt[...] = bc

    @pl.when(i > 0)
    def _():
        cnt[...] = cnt[...] + bc

    counts = cnt[...]
    ii = lax.broadcasted_iota(jnp.int32, (G, G), 0)
    jj = lax.broadcasted_iota(jnp.int32, (G, G), 1)
    tri = (ii < jj).astype(jnp.float32)
    offs = jnp.dot(counts, tri, preferred_element_type=jnp.float32)
    offs_ref[...] = jnp.concatenate(
        [offs, jnp.full((1, 16), float(N), jnp.float32)], axis=1)


def _k4_body(ws_ref, wm_ref, add_ref, wg1_ref, wg2_ref, bgo_ref,
             wf1_ref, wf2_ref, bf_ref, out_ref):
    gmat = (jnp.dot(ws_ref[...], wg1_ref[...], preferred_element_type=jnp.float32)
            + jnp.dot(wm_ref[...], wg2_ref[...], preferred_element_type=jnp.float32)
            + bgo_ref[...])
    out_ref[...] = (
        jnp.dot(gmat, wf1_ref[...], preferred_element_type=jnp.float32)
        + jnp.dot(add_ref[...], wf2_ref[...], preferred_element_type=jnp.float32)
        + bf_ref[...])


def _row_spec(blk, width):
    return pl.BlockSpec((blk, width), lambda i: (i, 0))


def _full_spec(shape):
    return pl.BlockSpec(shape, lambda *i: (0,) * len(shape))


def kernel(x, edge_index, batch, additional_input, W1, b1, W2, b2,
           w_pool, b_pool, W_go, b_go, W_f, b_f):
    src = edge_index[0]
    dst = edge_index[1]
    pad = E_PAD - E
    pad_idx = jnp.arange(pad, dtype=jnp.int32)
    src_p = jnp.concatenate([src, pad_idx % N]).reshape(NCH, CHUNK)
    dst_p = jnp.concatenate([dst, N + pad_idx % (NROW - N)]).reshape(NCH, CHUNK)

    rpad = NROW - N
    xp = jnp.pad(x, ((0, rpad), (0, 0)))
    batchp = jnp.pad(batch.reshape(N, 1), ((0, rpad), (0, 0)),
                     constant_values=G + 63)

    nsteps = NROW // _BLK
    xl1 = pl.pallas_call(
        _k1a_body,
        grid=(nsteps,),
        in_specs=[_row_spec(_BLK, D), _full_spec((D, D))],
        out_specs=_row_spec(_BLK, D),
        out_shape=jax.ShapeDtypeStruct((NROW, D), jnp.float32),
    )(xp, W1)

    degp = _deg_kernel(dst_p)

    y1, dinv = pl.pallas_call(
        _k1b_body,
        grid=(nsteps,),
        in_specs=[_row_spec(_BLK, D), _row_spec(_BLK, 1), _row_spec(_BLK, 1)],
        out_specs=[_row_spec(_BLK, D), _row_spec(_BLK, 1)],
        out_shape=[jax.ShapeDtypeStruct((NROW, D), jnp.float32),
                   jax.ShapeDtypeStruct((NROW, 1), jnp.float32)],
    )(xl1, degp[0, :, 0:1], degp[1, :, 0:1])

    acc1 = _scatter_kernel(y1, src_p, dst_p)

    xl2, y2 = pl.pallas_call(
        _k2_body,
        grid=(nsteps,),
        in_specs=[_row_spec(_BLK, D), _row_spec(_BLK, D), _row_spec(_BLK, D),
                  _row_spec(_BLK, 1), _full_spec((D, D)), _full_spec((1, D))],
        out_specs=[_row_spec(_BLK, D), _row_spec(_BLK, D)],
        out_shape=[jax.ShapeDtypeStruct((NROW, D), jnp.float32),
                   jax.ShapeDtypeStruct((NROW, D), jnp.float32)],
    )(acc1[0], acc1[1], xl1, dinv, W2, b1.reshape(1, D))

    acc2 = _scatter_kernel(y2, src_p, dst_p)

    h2, hw, offs = pl.pallas_call(
        _k3_body,
        grid=(nsteps,),
        in_specs=[_row_spec(_BLK, D), _row_spec(_BLK, D), _row_spec(_BLK, D),
                  _row_spec(_BLK, 1), _full_spec((1, D)), _full_spec((D, 1)),
                  _full_spec((1, 1)), _row_spec(_BLK, 1)],
        out_specs=[_row_spec(_BLK, D), _row_spec(_BLK, D), _full_spec((1, 80))],
        out_shape=[jax.ShapeDtypeStruct((NROW, D), jnp.float32),
                   jax.ShapeDtypeStruct((NROW, D), jnp.float32),
                   jax.ShapeDtypeStruct((1, 80), jnp.float32)],
        scratch_shapes=[pltpu.VMEM((1, G), jnp.float32)],
    )(acc2[0], acc2[1], xl2, dinv, b2.reshape(1, D), w_pool,
      b_pool.reshape(1, 1), batchp)

    wsum8, wmax8 = _pool_kernel(h2, hw, offs.reshape(80))
    wsum = wsum8.reshape(NTILES, 8, D)[:, :2].reshape(G, D)
    wmax = wmax8.reshape(NTILES, 8, D)[:, :2].reshape(G, D)

    out = pl.pallas_call(
        _k4_body,
        in_specs=[_full_spec((G, D)), _full_spec((G, D)), _full_spec((G, 16)),
                  _full_spec((D, 256)), _full_spec((D, 256)), _full_spec((1, 256)),
                  _full_spec((256, 1)), _full_spec((16, 1)), _full_spec((1, 1))],
        out_specs=_full_spec((G, 1)),
        out_shape=jax.ShapeDtypeStruct((G, 1), jnp.float32),
    )(wsum, wmax, additional_input, W_go[:D], W_go[D:], b_go.reshape(1, 256),
      W_f[:256], W_f[256:], b_f.reshape(1, 1))

    return out


# speedup vs baseline: 25.2767x; 1.0209x over previous
import functools

import jax
import jax.numpy as jnp
from jax import lax
from jax.experimental import pallas as pl
from jax.experimental.pallas import tpu as pltpu
from jax.experimental.pallas import tpu_sc as plsc

N = 10000
E = 320000
D = 128
G = 64
CHUNK = 128
NTILES = 32
CPT = 80
NCH = NTILES * CPT
E_PAD = NCH * CHUNK
NROW = 10240
RPT = NROW // 16

_SC_MESH = plsc.VectorSubcoreMesh(core_axis_name="c", subcore_axis_name="s")


@functools.partial(
    pl.kernel,
    out_type=jax.ShapeDtypeStruct((2, NROW, D), jnp.float32),
    mesh=_SC_MESH,
    scratch_types=[
        pltpu.VMEM((CPT, CHUNK), jnp.int32),
        pltpu.VMEM((CHUNK, D), jnp.float32),
        pltpu.VMEM_SHARED((NROW, D), jnp.float32),
    ],
)
def _deg_kernel(dst_hbm, out_hbm, dst_buf, ones_buf, acc):
    c = lax.axis_index("c")
    s = lax.axis_index("s")
    tile = c * 16 + s

    def zrow(i, _):
        for k in range(D // 16):
            ones_buf[i, pl.ds(k * 16, 16)] = jnp.zeros((16,), jnp.float32)
        return 0
    lax.fori_loop(0, CHUNK, zrow, 0)

    r0 = s * RPT
    for k in range(RPT // CHUNK):
        pltpu.sync_copy(ones_buf, acc.at[pl.ds(r0 + k * CHUNK, CHUNK)])
    rem = RPT % CHUNK
    if rem:
        pltpu.sync_copy(ones_buf.at[pl.ds(0, rem)],
                        acc.at[pl.ds(r0 + (RPT // CHUNK) * CHUNK, rem)])

    def orow(i, _):
        for k in range(D // 16):
            ones_buf[i, pl.ds(k * 16, 16)] = jnp.ones((16,), jnp.float32)
        return 0
    lax.fori_loop(0, CHUNK, orow, 0)

    pltpu.sync_copy(dst_hbm.at[pl.ds(tile * CPT, CPT)], dst_buf)
    plsc.subcore_barrier()

    def body(j, _):
        pltpu.sync_copy(ones_buf, acc.at[dst_buf.at[j]], add=True)
        return 0
    lax.fori_loop(0, CPT, body, 0)

    plsc.subcore_barrier()
    pltpu.sync_copy(acc.at[pl.ds(r0, RPT)], out_hbm.at[c, pl.ds(r0, RPT)])


NB = 40


@functools.partial(
    pl.kernel,
    out_type=jax.ShapeDtypeStruct((2, NROW, D), jnp.float32),
    mesh=_SC_MESH,
    scratch_types=[
        pltpu.VMEM((NB, CHUNK), jnp.int32),
        pltpu.VMEM((NB, CHUNK), jnp.int32),
        pltpu.VMEM((CHUNK, D), jnp.float32),
        pltpu.VMEM((CHUNK, D), jnp.float32),
        pltpu.VMEM_SHARED((NROW, D), jnp.float32),
        pltpu.SemaphoreType.DMA,
        pltpu.SemaphoreType.DMA,
    ],
)
def _scatter_kernel(y_hbm, src_hbm, dst_hbm, out_hbm,
                    src_buf, dst_buf, rows0, rows1, acc, sem0, sem1):
    c = lax.axis_index("c")
    s = lax.axis_index("s")
    tile = c * 16 + s
    bufs = (rows0, rows1)
    sems = (sem0, sem1)

    def zrow(i, _):
        for k in range(D // 16):
            rows0[i, pl.ds(k * 16, 16)] = jnp.zeros((16,), jnp.float32)
        return 0
    lax.fori_loop(0, CHUNK, zrow, 0)

    r0 = s * RPT
    for k in range(RPT // CHUNK):
        pltpu.sync_copy(rows0, acc.at[pl.ds(r0 + k * CHUNK, CHUNK)])
    rem = RPT % CHUNK
    if rem:
        pltpu.sync_copy(rows0.at[pl.ds(0, rem)],
                        acc.at[pl.ds(r0 + (RPT // CHUNK) * CHUNK, rem)])
    plsc.subcore_barrier()

    def wait_gather(b):
        pltpu.make_async_copy(y_hbm.at[pl.ds(0, CHUNK)], bufs[b], sems[b]).wait()

    for blk in range(CPT // NB):
        base = tile * CPT + blk * NB
        pltpu.sync_copy(src_hbm.at[pl.ds(base, NB)], src_buf)
        pltpu.sync_copy(dst_hbm.at[pl.ds(base, NB)], dst_buf)
        for b in range(2):
            pltpu.async_copy(y_hbm.at[src_buf.at[b]], bufs[b], sems[b])

        def pair_body(jj, _):
            for b in range(2):
                j = jj * 2 + b
                wait_gather(b)
                pltpu.sync_copy(bufs[b], acc.at[dst_buf.at[j]], add=True)

                @pl.when(j + 2 < NB)
                def _():
                    pltpu.async_copy(y_hbm.at[src_buf.at[j + 2]],
                                     bufs[b], sems[b])
            return 0
        lax.fori_loop(0, NB // 2, pair_body, 0)

    plsc.subcore_barrier()
    pltpu.sync_copy(acc.at[pl.ds(r0, RPT)], out_hbm.at[c, pl.ds(r0, RPT)])


@functools.partial(
    pl.kernel,
    out_type=jax.ShapeDtypeStruct((NTILES * 8, D), jnp.float32),
    mesh=_SC_MESH,
    scratch_types=[
        pltpu.VMEM((80,), jnp.float32),
        pltpu.VMEM((CHUNK, D), jnp.float32),
        pltpu.VMEM((8, D), jnp.float32),
    ],
)
def _pool_kernel(h2_hbm, offs_hbm, wmax_hbm, offs_buf, bufa, mrows):
    c = lax.axis_index("c")
    s = lax.axis_index("s")
    wid = c * 16 + s
    pltpu.sync_copy(offs_hbm, offs_buf)
    nv = D // 16
    for lg in range(2):
        g = wid * 2 + lg
        ovec = offs_buf[pl.ds(g, 16)]
        start = ovec[0].astype(jnp.int32)
        end = ovec[1].astype(jnp.int32)
        start_al = pl.multiple_of((start // 8) * 8, 8)
        shift = start - start_al
        count = end - start_al
        nchunks = (count + CHUNK - 1) // CHUNK

        def chunk_body(i, accm):
            off = pl.multiple_of(start_al + i * CHUNK, 8)
            pltpu.sync_copy(h2_hbm.at[pl.ds(off, CHUNK)], bufa)
            lo = jnp.where(i == 0, shift, 0)
            hi = jnp.minimum(count - i * CHUNK, CHUNK)

            def row_body(j, a_m):
                return tuple(jnp.maximum(a_m[k], bufa[j, pl.ds(k * 16, 16)])
                             for k in range(nv))
            return lax.fori_loop(lo, hi, row_body, accm)

        init = tuple(jnp.full((16,), -jnp.inf, jnp.float32)
                     for _ in range(nv))
        accm = lax.fori_loop(0, nchunks, chunk_body, init)
        for k in range(nv):
            mrows[lg, pl.ds(k * 16, 16)] = accm[k]
    pltpu.sync_copy(mrows, wmax_hbm.at[pl.ds(wid * 8, 8)])


_BLK = 1024


def _k1a_body(x_ref, w1_ref, xl_ref):
    xl_ref[...] = jnp.dot(x_ref[...], w1_ref[...],
                          preferred_element_type=jnp.float32)


def _k1b_body(xl_ref, d0_ref, d1_ref, y_ref, dinv_ref):
    deg = d0_ref[...] + d1_ref[...] + 1.0
    dinv = lax.rsqrt(deg)
    y_ref[...] = xl_ref[...] * dinv
    dinv_ref[...] = dinv


def _k2_body(a0_ref, a1_ref, xl1_ref, dinv_ref, w2_ref, b1_ref,
             xl2_ref, y2_ref):
    dv = dinv_ref[...]
    h1 = dv * (a0_ref[...] + a1_ref[...]) + dv * dv * xl1_ref[...] + b1_ref[...]
    xl2 = jnp.dot(h1, w2_ref[...], preferred_element_type=jnp.float32)
    xl2_ref[...] = xl2
    y2_ref[...] = xl2 * dv


def _k3_body(a0_ref, a1_ref, xl2_ref, dinv_ref, b2_ref, wp_ref, bp_ref,
             batch_ref, h2_ref, offs_ref, wsum_ref, cnt, wacc):
    i = pl.program_id(0)
    dv = dinv_ref[...]
    h2 = dv * (a0_ref[...] + a1_ref[...]) + dv * dv * xl2_ref[...] + b2_ref[...]
    w = jax.nn.sigmoid(
        jnp.dot(h2, wp_ref[...], preferred_element_type=jnp.float32)
        + bp_ref[...])
    h2_ref[...] = h2
    hw = h2 * w
    gids = lax.broadcasted_iota(jnp.int32, (1, G), 1)
    oh = (batch_ref[...] == gids).astype(jnp.float32)
    bc = jnp.sum(oh, axis=0, keepdims=True)
    ws = lax.dot_general(oh, hw, (((0,), (0,)), ((), ())),
                         preferred_element_type=jnp.float32)

    @pl.when(i == 0)
    def _():
        cnt[...] = bc
        wacc[...] = ws

    @pl.when(i > 0)
    def _():
        cnt[...] = cnt[...] + bc
        wacc[...] = wacc[...] + ws

    wsum_ref[...] = wacc[...]
    counts = cnt[...]
    ii = lax.broadcasted_iota(jnp.int32, (G, G), 0)
    jj = lax.broadcasted_iota(jnp.int32, (G, G), 1)
    tri = (ii < jj).astype(jnp.float32)
    offs = jnp.dot(counts, tri, preferred_element_type=jnp.float32)
    offs_ref[...] = jnp.concatenate(
        [offs, jnp.full((1, 16), float(N), jnp.float32)], axis=1)


def _k4_body(ws_ref, wm_ref, add_ref, wg1_ref, wg2_ref, bgo_ref,
             wf1_ref, wf2_ref, bf_ref, out_ref):
    gmat = (jnp.dot(ws_ref[...], wg1_ref[...], preferred_element_type=jnp.float32)
            + jnp.dot(wm_ref[...], wg2_ref[...], preferred_element_type=jnp.float32)
            + bgo_ref[...])
    out_ref[...] = (
        jnp.dot(gmat, wf1_ref[...], preferred_element_type=jnp.float32)
        + jnp.dot(add_ref[...], wf2_ref[...], preferred_element_type=jnp.float32)
        + bf_ref[...])


def _row_spec(blk, width):
    return pl.BlockSpec((blk, width), lambda i: (i, 0))


def _full_spec(shape):
    return pl.BlockSpec(shape, lambda *i: (0,) * len(shape))


def kernel(x, edge_index, batch, additional_input, W1, b1, W2, b2,
           w_pool, b_pool, W_go, b_go, W_f, b_f):
    src = edge_index[0]
    dst = edge_index[1]
    pad = E_PAD - E
    pad_idx = jnp.arange(pad, dtype=jnp.int32)
    src_p = jnp.concatenate([src, pad_idx % N]).reshape(NCH, CHUNK)
    dst_p = jnp.concatenate([dst, N + pad_idx % (NROW - N)]).reshape(NCH, CHUNK)

    rpad = NROW - N
    xp = jnp.pad(x, ((0, rpad), (0, 0)))
    batchp = jnp.pad(batch.reshape(N, 1), ((0, rpad), (0, 0)),
                     constant_values=G + 63)

    nsteps = NROW // _BLK
    xl1 = pl.pallas_call(
        _k1a_body,
        grid=(nsteps,),
        in_specs=[_row_spec(_BLK, D), _full_spec((D, D))],
        out_specs=_row_spec(_BLK, D),
        out_shape=jax.ShapeDtypeStruct((NROW, D), jnp.float32),
    )(xp, W1)

    degp = _deg_kernel(dst_p)

    y1, dinv = pl.pallas_call(
        _k1b_body,
        grid=(nsteps,),
        in_specs=[_row_spec(_BLK, D), _row_spec(_BLK, 1), _row_spec(_BLK, 1)],
        out_specs=[_row_spec(_BLK, D), _row_spec(_BLK, 1)],
        out_shape=[jax.ShapeDtypeStruct((NROW, D), jnp.float32),
                   jax.ShapeDtypeStruct((NROW, 1), jnp.float32)],
    )(xl1, degp[0, :, 0:1], degp[1, :, 0:1])

    acc1 = _scatter_kernel(y1, src_p, dst_p)

    xl2, y2 = pl.pallas_call(
        _k2_body,
        grid=(nsteps,),
        in_specs=[_row_spec(_BLK, D), _row_spec(_BLK, D), _row_spec(_BLK, D),
                  _row_spec(_BLK, 1), _full_spec((D, D)), _full_spec((1, D))],
        out_specs=[_row_spec(_BLK, D), _row_spec(_BLK, D)],
        out_shape=[jax.ShapeDtypeStruct((NROW, D), jnp.float32),
                   jax.ShapeDtypeStruct((NROW, D), jnp.float32)],
    )(acc1[0], acc1[1], xl1, dinv, W2, b1.reshape(1, D))

    acc2 = _scatter_kernel(y2, src_p, dst_p)

    h2, offs, wsum = pl.pallas_call(
        _k3_body,
        grid=(nsteps,),
        in_specs=[_row_spec(_BLK, D), _row_spec(_BLK, D), _row_spec(_BLK, D),
                  _row_spec(_BLK, 1), _full_spec((1, D)), _full_spec((D, 1)),
                  _full_spec((1, 1)), _row_spec(_BLK, 1)],
        out_specs=[_row_spec(_BLK, D), _full_spec((1, 80)),
                   _full_spec((G, D))],
        out_shape=[jax.ShapeDtypeStruct((NROW, D), jnp.float32),
                   jax.ShapeDtypeStruct((1, 80), jnp.float32),
                   jax.ShapeDtypeStruct((G, D), jnp.float32)],
        scratch_shapes=[pltpu.VMEM((1, G), jnp.float32),
                        pltpu.VMEM((G, D), jnp.float32)],
    )(acc2[0], acc2[1], xl2, dinv, b2.reshape(1, D), w_pool,
      b_pool.reshape(1, 1), batchp)

    wmax8 = _pool_kernel(h2, offs.reshape(80))
    wmax = wmax8.reshape(NTILES, 8, D)[:, :2].reshape(G, D)

    out = pl.pallas_call(
        _k4_body,
        in_specs=[_full_spec((G, D)), _full_spec((G, D)), _full_spec((G, 16)),
                  _full_spec((D, 256)), _full_spec((D, 256)), _full_spec((1, 256)),
                  _full_spec((256, 1)), _full_spec((16, 1)), _full_spec((1, 1))],
        out_specs=_full_spec((G, 1)),
        out_shape=jax.ShapeDtypeStruct((G, 1), jnp.float32),
    )(wsum, wmax, additional_input, W_go[:D], W_go[D:], b_go.reshape(1, 256),
      W_f[:256], W_f[256:], b_f.reshape(1, 1))

    return out
